# baseline (speedup 1.0000x reference)
import functools

import jax
import jax.numpy as jnp
from jax import lax
from jax.experimental import pallas as pl
from jax.experimental.pallas import tpu as pltpu
from jax.experimental.pallas import tpu_sc as plsc

A, H, W, CH = 3, 256, 256, 85
NPOS = A * H * W
N = 4096
NCLS = 80
NC, NS = 2, 16
NW = NC * NS
GPW = N // NW
SPT = N // NS
ZPT = NPOS // NS
EPS = 1e-07


def _sc_body(pred1d, idx2, widx3, ones_h, zeros_h, gath_out, obj_out,
             widx_v, rows_v, sidx_v, ones_v, zeros_v, sem):
    c = lax.axis_index("c")
    s = lax.axis_index("s")
    wid = s * NC + c
    pltpu.sync_copy(widx3.at[wid], widx_v)
    handles = [pltpu.async_copy(pred1d.at[widx_v.at[j]], rows_v.at[j], sem)
               for j in range(CH)]
    for h in handles:
        h.wait()
    pltpu.sync_copy(rows_v, gath_out.at[wid])

    @pl.when(c == 0)
    def _():
        pltpu.sync_copy(zeros_h, zeros_v)
        pltpu.sync_copy(zeros_v, obj_out.at[pl.ds(s * ZPT, ZPT)])
        plsc.subcore_barrier()
        pltpu.sync_copy(ones_h, ones_v)
        for j in range(SPT // GPW):
            pltpu.sync_copy(idx2.at[s * (SPT // GPW) + j], sidx_v)
            pltpu.async_copy(ones_v, obj_out.at[sidx_v], sem).wait()


@functools.lru_cache(maxsize=1)
def _get_sc_call():
    return pl.kernel(
        _sc_body,
        out_type=[
            jax.ShapeDtypeStruct((NW, CH, GPW), jnp.float32),
            jax.ShapeDtypeStruct((NPOS,), jnp.float32),
        ],
        mesh=plsc.VectorSubcoreMesh(core_axis_name="c", subcore_axis_name="s",
                                    num_cores=NC, num_subcores=NS),
        compiler_params=pltpu.CompilerParams(use_tc_tiling_on_sc=False),
        scratch_types=[
            pltpu.VMEM((CH, GPW), jnp.int32),
            pltpu.VMEM((CH, GPW), jnp.float32),
            pltpu.VMEM((GPW,), jnp.int32),
            pltpu.VMEM((GPW,), jnp.float32),
            pltpu.VMEM((ZPT,), jnp.float32),
            pltpu.SemaphoreType.DMA,
        ],
    )


OBR = 64


def _objbase_body(p_ref, sum_ref, xcol_ref):
    i = pl.program_id(0)

    @pl.when(i == 0)
    def _():
        sum_ref[...] = jnp.zeros_like(sum_ref)

    x = p_ref[:, :, 4]
    xcol_ref[...] = x
    t = jnp.maximum(x, 0.0) + jnp.log(1.0 + jnp.exp(-jnp.abs(x)))
    sum_ref[...] += jnp.sum(t).reshape(1, 1)


def _objcorr_body(x_ref, z_ref, out_ref):
    out_ref[...] = jnp.sum(x_ref[...] * z_ref[...]).reshape(1, 1)


def _cls_body(x_ref, z_ref, out_ref):
    x = x_ref[...]
    z = z_ref[...]
    t = jnp.maximum(x, 0.0) - x * z + jnp.log(1.0 + jnp.exp(-jnp.abs(x)))
    out_ref[...] = jnp.sum(t).reshape(1, 1)


def _atan(t):
    a = jnp.abs(t)
    inv = a > 1.0
    u = jnp.where(inv, 1.0 / a, a)
    u2 = u * u
    p = u * (0.9998660 + u2 * (-0.3302995 + u2 * (0.1801410
             + u2 * (-0.0851330 + u2 * 0.0208351))))
    r = jnp.where(inv, 1.5707964 - p, p)
    return jnp.where(t < 0.0, -r, r)


TI = 512
TJ = 512


def _ciou_body(pr_ref, bt_ref, out_ref):
    i = pl.program_id(0)
    j = pl.program_id(1)

    @pl.when((i == 0) & (j == 0))
    def _():
        out_ref[...] = jnp.zeros_like(out_ref)

    b1x1 = pr_ref[:, 0:1]
    b1y1 = pr_ref[:, 1:2]
    b1x2 = pr_ref[:, 2:3]
    b1y2 = pr_ref[:, 3:4]
    b2x1 = bt_ref[0:1, :]
    b2y1 = bt_ref[1:2, :]
    b2x2 = bt_ref[2:3, :]
    b2y2 = bt_ref[3:4, :]

    inter_x1 = jnp.maximum(b1x1, b2x1)
    inter_y1 = jnp.maximum(b1y1, b2y1)
    inter_x2 = jnp.minimum(b1x2, b2x2)
    inter_y2 = jnp.minimum(b1y2, b2y2)
    inter = (jnp.clip(inter_x2 - inter_x1, 0.0)
             * jnp.clip(inter_y2 - inter_y1, 0.0))
    area1 = (b1x2 - b1x1) * (b1y2 - b1y1)
    area2 = (b2x2 - b2x1) * (b2y2 - b2y1)
    union = area1 + area2 - inter
    iou = inter / (union + EPS)
    enc_w = jnp.maximum(b1x2, b2x2) - jnp.minimum(b1x1, b2x1)
    enc_h = jnp.maximum(b1y2, b2y2) - jnp.minimum(b1y1, b2y1)
    diag2 = enc_w * enc_w + enc_h * enc_h + EPS
    dist2 = (((b1x1 + b1x2) - (b2x1 + b2x2)) ** 2
             + ((b1y1 + b1y2) - (b2y1 + b2y2)) ** 2) * 0.25
    angle1 = _atan((b1x2 - b1x1) / (b1y2 - b1y1 + EPS))
    angle2 = _atan((b2x2 - b2x1) / (b2y2 - b2y1 + EPS))
    d = angle1 - angle2
    v = (4.0 / 3.1416 ** 2) * d * d
    alpha = v / ((1.0 - iou) + v + EPS)
    ciou = iou - dist2 / diag2 - alpha * v
    out_ref[...] += jnp.sum(ciou).reshape(1, 1)


def kernel(pred, box, cls, grid_x, grid_y, grid_anchor):
    f32 = jnp.float32
    flat = (grid_anchor.astype(jnp.int32) * (H * W)
            + grid_y.astype(jnp.int32) * W + grid_x.astype(jnp.int32))
    idx2 = flat.reshape(NW, GPW)
    widx3 = (flat[:, None] * CH
             + jnp.arange(CH, dtype=jnp.int32)[None, :]).reshape(NW, CH, GPW)
    pred1d = pred.reshape(NPOS * CH)
    ones_h = jnp.ones((GPW,), f32)
    zeros_h = jnp.zeros((ZPT,), f32)

    gath3, mask = _get_sc_call()(pred1d, idx2, widx3, ones_h, zeros_h)
    gath = gath3.reshape(N, CH)

    pred3d = pred.reshape(NPOS // 128, 128, CH)
    NB = NPOS // 128 // OBR
    s_base, xcol = pl.pallas_call(
        _objbase_body,
        grid=(NB,),
        in_specs=[pl.BlockSpec((OBR, 128, CH), lambda i: (i, 0, 0))],
        out_specs=[
            pl.BlockSpec((1, 1), lambda i: (0, 0)),
            pl.BlockSpec((OBR, 128), lambda i: (i, 0)),
        ],
        out_shape=[
            jax.ShapeDtypeStruct((1, 1), f32),
            jax.ShapeDtypeStruct((NPOS // 128, 128), f32),
        ],
    )(pred3d)
    mask2 = mask.reshape(NPOS // 128, 128)
    s_corr = pl.pallas_call(
        _objcorr_body,
        out_shape=jax.ShapeDtypeStruct((1, 1), f32),
    )(xcol, mask2)

    pr = gath[:, 0:4]
    boxt = jnp.zeros((8, N), f32).at[0:4, :].set(box.T)
    s_ciou = pl.pallas_call(
        _ciou_body,
        grid=(N // TI, N // TJ),
        in_specs=[
            pl.BlockSpec((TI, 4), lambda i, j: (i, 0)),
            pl.BlockSpec((8, TJ), lambda i, j: (0, j)),
        ],
        out_specs=pl.BlockSpec((1, 1), lambda i, j: (0, 0)),
        out_shape=jax.ShapeDtypeStruct((1, 1), f32),
    )(pr, boxt)

    s_cls = pl.pallas_call(
        _cls_body,
        out_shape=jax.ShapeDtypeStruct((1, 1), f32),
    )(gath[:, 5:CH], cls)

    loss_obj = (s_base[0, 0] - s_corr[0, 0]) / NPOS
    loss_box = 1.0 - s_ciou[0, 0] / (N * N)
    loss_cls = s_cls[0, 0] / (N * NCLS)
    return loss_obj + loss_box + loss_cls


# speedup vs baseline: 1.1640x; 1.1640x over previous
import functools

import jax
import jax.numpy as jnp
from jax import lax
from jax.experimental import pallas as pl
from jax.experimental.pallas import tpu as pltpu
from jax.experimental.pallas import tpu_sc as plsc

A, H, W, CH = 3, 256, 256, 85
NPOS = A * H * W
N = 4096
NCLS = 80
NC, NS = 2, 16
NW = NC * NS
GPW = N // NW
SPT = N // NS
ZPT = NPOS // NS
EPS = 1e-07


def _sc_body(pred1d, idx2, widx3, ones_h, zeros_h, gath_out, obj_out,
             widx_v, rows_v, sidx_v, ones_v, zeros_v, sem):
    c = lax.axis_index("c")
    s = lax.axis_index("s")
    wid = s * NC + c
    pltpu.sync_copy(widx3.at[wid], widx_v)
    handles = [pltpu.async_copy(pred1d.at[widx_v.at[j]], rows_v.at[j], sem)
               for j in range(CH)]
    for h in handles:
        h.wait()
    pltpu.sync_copy(rows_v, gath_out.at[wid])

    @pl.when(c == 0)
    def _():
        pltpu.sync_copy(zeros_h, zeros_v)
        pltpu.sync_copy(zeros_v, obj_out.at[pl.ds(s * ZPT, ZPT)])
        plsc.subcore_barrier()
        pltpu.sync_copy(ones_h, ones_v)
        for j in range(SPT // GPW):
            pltpu.sync_copy(idx2.at[s * (SPT // GPW) + j], sidx_v)
            pltpu.async_copy(ones_v, obj_out.at[sidx_v], sem).wait()


@functools.lru_cache(maxsize=1)
def _get_sc_call():
    return pl.kernel(
        _sc_body,
        out_type=[
            jax.ShapeDtypeStruct((NW, CH, GPW), jnp.float32),
            jax.ShapeDtypeStruct((NPOS,), jnp.float32),
        ],
        mesh=plsc.VectorSubcoreMesh(core_axis_name="c", subcore_axis_name="s",
                                    num_cores=NC, num_subcores=NS),
        compiler_params=pltpu.CompilerParams(use_tc_tiling_on_sc=False),
        scratch_types=[
            pltpu.VMEM((CH, GPW), jnp.int32),
            pltpu.VMEM((CH, GPW), jnp.float32),
            pltpu.VMEM((GPW,), jnp.int32),
            pltpu.VMEM((GPW,), jnp.float32),
            pltpu.VMEM((ZPT,), jnp.float32),
            pltpu.SemaphoreType.DMA,
        ],
    )


TI = 512
TJ = 512
INV_NPOS = 1.0 / NPOS
INV_CLS = 1.0 / (N * NCLS)
INV_NN = 1.0 / (N * N)
ANG_S = 2.0 / 3.1416


def _atan(t):
    a = jnp.abs(t)
    inv = a > 1.0
    u = jnp.where(inv, 1.0 / a, a)
    u2 = u * u
    p = u * (0.9998660 + u2 * (-0.3302995 + u2 * (0.1801410
             + u2 * (-0.0851330 + u2 * 0.0208351))))
    r = jnp.where(inv, 1.5707964 - p, p)
    return jnp.where(t < 0.0, -r, r)


def _bce_sum(x, z):
    return jnp.sum(jnp.maximum(x, 0.0) - x * z
                   + jnp.log(1.0 + jnp.exp(-jnp.abs(x))))


def _fused_body(ox_ref, mz_ref, cx_ref, cz_ref, pr_ref, bt_ref, out_ref):
    k = pl.program_id(0)

    @pl.when(k == 0)
    def _():
        s_obj = _bce_sum(ox_ref[...], mz_ref[...])
        s_cls = _bce_sum(cx_ref[...], cz_ref[...])
        out_ref[...] = (s_obj * INV_NPOS + s_cls * INV_CLS
                        + 1.0).reshape(1, 1)

    @pl.when(k > 0)
    def _():
        x1c = pr_ref[:, 0:1]
        y1c = pr_ref[:, 1:2]
        x2c = pr_ref[:, 2:3]
        y2c = pr_ref[:, 3:4]
        x1r = bt_ref[0:1, :]
        y1r = bt_ref[1:2, :]
        x2r = bt_ref[2:3, :]
        y2r = bt_ref[3:4, :]
        wc = x2c - x1c
        hc = y2c - y1c
        wr = x2r - x1r
        hr = y2r - y1r
        areac = wc * hc
        arear = wr * hr
        cxc = (x1c + x2c) * 0.5
        cyc = (y1c + y2c) * 0.5
        cxr = (x1r + x2r) * 0.5
        cyr = (y1r + y2r) * 0.5
        angc = ANG_S * _atan(wc / (hc + EPS))
        angr = ANG_S * _atan(wr / (hr + EPS))

        ix1 = jnp.maximum(x1c, x1r)
        iy1 = jnp.maximum(y1c, y1r)
        ix2 = jnp.minimum(x2c, x2r)
        iy2 = jnp.minimum(y2c, y2r)
        iw = ix2 - ix1
        ih = iy2 - iy1
        inter = jnp.maximum(iw, 0.0) * jnp.maximum(ih, 0.0)
        union = (areac + arear) - inter
        iou = inter / (union + EPS)
        encw = (wc + wr) - iw
        ench = (hc + hr) - ih
        diag2 = encw * encw + (ench * ench + EPS)
        dx = cxc - cxr
        dy = cyc - cyr
        dist2 = dx * dx + dy * dy
        dv = angc - angr
        v = dv * dv
        denom = ((1.0 - iou) + v) + EPS
        ciou = (iou - dist2 / diag2) - (v / denom) * v
        out_ref[...] += (jnp.sum(ciou) * (-INV_NN)).reshape(1, 1)


def kernel(pred, box, cls, grid_x, grid_y, grid_anchor):
    f32 = jnp.float32
    flat = (grid_anchor.astype(jnp.int32) * (H * W)
            + grid_y.astype(jnp.int32) * W + grid_x.astype(jnp.int32))
    idx2 = flat.reshape(NW, GPW)
    widx3 = (flat[:, None] * CH
             + jnp.arange(CH, dtype=jnp.int32)[None, :]).reshape(NW, CH, GPW)
    pred1d = pred.reshape(NPOS * CH)
    ones_h = jnp.ones((GPW,), f32)
    zeros_h = jnp.zeros((ZPT,), f32)

    gath3, mask = _get_sc_call()(pred1d, idx2, widx3, ones_h, zeros_h)
    gath = gath3.reshape(N, CH)

    obj_x = pred[..., 4].reshape(NPOS // 128, 128)
    mask2 = mask.reshape(NPOS // 128, 128)
    boxt = jnp.zeros((8, N), f32).at[0:4, :].set(box.T)

    njt = N // TJ
    total = pl.pallas_call(
        _fused_body,
        grid=(1 + (N // TI) * njt,),
        in_specs=[
            pl.BlockSpec((NPOS // 128, 128), lambda k: (0, 0)),
            pl.BlockSpec((NPOS // 128, 128), lambda k: (0, 0)),
            pl.BlockSpec((N, NCLS), lambda k: (0, 0)),
            pl.BlockSpec((N, NCLS), lambda k: (0, 0)),
            pl.BlockSpec((TI, 4), lambda k: (jnp.maximum(k - 1, 0) // njt, 0)),
            pl.BlockSpec((8, TJ), lambda k: (0, jnp.maximum(k - 1, 0) % njt)),
        ],
        out_specs=pl.BlockSpec((1, 1), lambda k: (0, 0)),
        out_shape=jax.ShapeDtypeStruct((1, 1), f32),
    )(obj_x, mask2, gath[:, 5:CH], cls, gath[:, 0:4], boxt)
    return total[0, 0]


# speedup vs baseline: 1.2498x; 1.0737x over previous
import functools

import jax
import jax.numpy as jnp
from jax import lax
from jax.experimental import pallas as pl
from jax.experimental.pallas import tpu as pltpu
from jax.experimental.pallas import tpu_sc as plsc

A, H, W, CH = 3, 256, 256, 85
NPOS = A * H * W
N = 4096
NCLS = 80
NC, NS = 2, 16
NW = NC * NS
GPW = N // NW
SPT = N // NS
ZPT = NPOS // NS
EPS = 1e-07


def _sc_body(pred1d, idx2, widx3, ones_h, zeros_h, gath_out, obj_out,
             widx_v, rows_v, sidx_v, ones_v, zeros_v, sem, sem2):
    c = lax.axis_index("c")
    s = lax.axis_index("s")
    wid = s * NC + c
    pltpu.sync_copy(widx3.at[wid], widx_v)
    handles = [pltpu.async_copy(pred1d.at[widx_v.at[j]], rows_v.at[j], sem)
               for j in range(CH)]

    @pl.when(c == 0)
    def _():
        pltpu.sync_copy(zeros_h, zeros_v)
        pltpu.sync_copy(zeros_v, obj_out.at[pl.ds(s * ZPT, ZPT)])
        plsc.subcore_barrier()
        pltpu.sync_copy(ones_h, ones_v)
        for j in range(SPT // GPW):
            pltpu.sync_copy(idx2.at[s * (SPT // GPW) + j], sidx_v)
            pltpu.async_copy(ones_v, obj_out.at[sidx_v], sem2).wait()

    for h in handles:
        h.wait()
    pltpu.sync_copy(rows_v, gath_out.at[wid])


@functools.lru_cache(maxsize=1)
def _get_sc_call():
    return pl.kernel(
        _sc_body,
        out_type=[
            jax.ShapeDtypeStruct((NW, CH, GPW), jnp.float32),
            jax.ShapeDtypeStruct((NPOS,), jnp.float32),
        ],
        mesh=plsc.VectorSubcoreMesh(core_axis_name="c", subcore_axis_name="s",
                                    num_cores=NC, num_subcores=NS),
        compiler_params=pltpu.CompilerParams(use_tc_tiling_on_sc=False),
        scratch_types=[
            pltpu.VMEM((CH, GPW), jnp.int32),
            pltpu.VMEM((CH, GPW), jnp.float32),
            pltpu.VMEM((GPW,), jnp.int32),
            pltpu.VMEM((GPW,), jnp.float32),
            pltpu.VMEM((ZPT,), jnp.float32),
            pltpu.SemaphoreType.DMA,
            pltpu.SemaphoreType.DMA,
        ],
    )


TI = 512
TJ = 512
INV_NPOS = 1.0 / NPOS
INV_CLS = 1.0 / (N * NCLS)
INV_NN = 1.0 / (N * N)
ANG_S = 2.0 / 3.1416


def _atan(t):
    a = jnp.abs(t)
    inv = a > 1.0
    u = jnp.where(inv, 1.0 / a, a)
    u2 = u * u
    p = u * (0.9998660 + u2 * (-0.3302995 + u2 * (0.1801410
             + u2 * (-0.0851330 + u2 * 0.0208351))))
    r = jnp.where(inv, 1.5707964 - p, p)
    return jnp.where(t < 0.0, -r, r)


def _bce_sum(x, z):
    return jnp.sum(jnp.maximum(x, 0.0) - x * z
                   + jnp.log(1.0 + jnp.exp(-jnp.abs(x))))


def _prep(x1, y1, x2, y2):
    w = x2 - x1
    h = y2 - y1
    area = w * h
    cx = (x1 + x2) * 0.5
    cy = (y1 + y2) * 0.5
    ang = ANG_S * _atan(w / (h + EPS))
    return [x1, y1, x2, y2, w, h, area, cx, cy, ang]


NPAR = 10


def _fused_body(ox_ref, mz_ref, cx_ref, cz_ref, pr_ref, bt_ref, out_ref,
                *scr):
    pc = scr[:NPAR]
    pr_s = scr[NPAR:]
    k = pl.program_id(0)

    @pl.when(k == 0)
    def _():
        s_obj = _bce_sum(ox_ref[...], mz_ref[...])
        s_cls = _bce_sum(cx_ref[...], cz_ref[...])
        out_ref[...] = (s_obj * INV_NPOS + s_cls * INV_CLS
                        + 1.0).reshape(1, 1)
        cvals = _prep(pr_ref[:, 0:1], pr_ref[:, 1:2],
                      pr_ref[:, 2:3], pr_ref[:, 3:4])
        rvals = _prep(bt_ref[0:1, :], bt_ref[1:2, :],
                      bt_ref[2:3, :], bt_ref[3:4, :])
        for q in range(NPAR):
            pc[q][...] = cvals[q]
            pr_s[q][...] = rvals[q]

    @pl.when(k > 0)
    def _():
        kk = k - 1
        ib = kk // (N // TJ)
        jb = kk % (N // TJ)
        ci = ib * TI
        rj = jb * TJ
        (x1c, y1c, x2c, y2c, wc, hc, areac, cxc, cyc, angc) = [
            p[pl.ds(ci, TI), :] for p in pc]
        (x1r, y1r, x2r, y2r, wr, hr, arear, cxr, cyr, angr) = [
            p[:, pl.ds(rj, TJ)] for p in pr_s]

        ix1 = jnp.maximum(x1c, x1r)
        iy1 = jnp.maximum(y1c, y1r)
        ix2 = jnp.minimum(x2c, x2r)
        iy2 = jnp.minimum(y2c, y2r)
        iw = ix2 - ix1
        ih = iy2 - iy1
        inter = jnp.maximum(iw, 0.0) * jnp.maximum(ih, 0.0)
        union = (areac + arear) - inter
        iou = inter / (union + EPS)
        encw = (wc + wr) - iw
        ench = (hc + hr) - ih
        diag2 = encw * encw + (ench * ench + EPS)
        dx = cxc - cxr
        dy = cyc - cyr
        dist2 = dx * dx + dy * dy
        dv = angc - angr
        v = dv * dv
        denom = ((1.0 - iou) + v) + EPS
        ciou = (iou - dist2 / diag2) - (v / denom) * v
        out_ref[...] += (jnp.sum(ciou) * (-INV_NN)).reshape(1, 1)


def kernel(pred, box, cls, grid_x, grid_y, grid_anchor):
    f32 = jnp.float32
    flat = (grid_anchor.astype(jnp.int32) * (H * W)
            + grid_y.astype(jnp.int32) * W + grid_x.astype(jnp.int32))
    idx2 = flat.reshape(NW, GPW)
    widx3 = (flat[:, None] * CH
             + jnp.arange(CH, dtype=jnp.int32)[None, :]).reshape(NW, CH, GPW)
    pred1d = pred.reshape(NPOS * CH)
    ones_h = jnp.ones((GPW,), f32)
    zeros_h = jnp.zeros((ZPT,), f32)

    gath3, mask = _get_sc_call()(pred1d, idx2, widx3, ones_h, zeros_h)
    gath = gath3.reshape(N, CH)

    obj_x = pred[..., 4].reshape(NPOS // 128, 128)
    mask2 = mask.reshape(NPOS // 128, 128)
    boxt = jnp.zeros((8, N), f32).at[0:4, :].set(box.T)

    njt = N // TJ
    total = pl.pallas_call(
        _fused_body,
        grid=(1 + (N // TI) * njt,),
        in_specs=[
            pl.BlockSpec((NPOS // 128, 128), lambda k: (0, 0)),
            pl.BlockSpec((NPOS // 128, 128), lambda k: (0, 0)),
            pl.BlockSpec((N, NCLS), lambda k: (0, 0)),
            pl.BlockSpec((N, NCLS), lambda k: (0, 0)),
            pl.BlockSpec((N, 4), lambda k: (0, 0)),
            pl.BlockSpec((8, N), lambda k: (0, 0)),
        ],
        out_specs=pl.BlockSpec((1, 1), lambda k: (0, 0)),
        out_shape=jax.ShapeDtypeStruct((1, 1), f32),
        scratch_shapes=([pltpu.VMEM((N, 1), f32)] * 10
                        + [pltpu.VMEM((1, N), f32)] * 10),
    )(obj_x, mask2, gath[:, 5:CH], cls, gath[:, 0:4], boxt)
    return total[0, 0]


# speedup vs baseline: 1.2909x; 1.0328x over previous
import functools

import jax
import jax.numpy as jnp
from jax import lax
from jax.experimental import pallas as pl
from jax.experimental.pallas import tpu as pltpu
from jax.experimental.pallas import tpu_sc as plsc

A, H, W, CH = 3, 256, 256, 85
NPOS = A * H * W
N = 4096
NCLS = 80
NC, NS = 2, 16
NW = NC * NS
GPW = N // NW
SPT = N // NS
ZPT = NPOS // NS
EPS = 1e-07


def _sc_body(pred1d, idx2, widx3, ones_h, zeros_h, gath_out, obj_out,
             widx_v, rows_v, sidx_v, ones_v, zeros_v, sem, sem2):
    c = lax.axis_index("c")
    s = lax.axis_index("s")
    wid = s * NC + c
    pltpu.sync_copy(widx3.at[wid], widx_v)
    handles = [pltpu.async_copy(pred1d.at[widx_v.at[j]], rows_v.at[j], sem)
               for j in range(CH)]

    @pl.when(c == 0)
    def _():
        pltpu.sync_copy(zeros_h, zeros_v)
        pltpu.sync_copy(zeros_v, obj_out.at[pl.ds(s * ZPT, ZPT)])
        plsc.subcore_barrier()
        pltpu.sync_copy(ones_h, ones_v)
        for j in range(SPT // GPW):
            pltpu.sync_copy(idx2.at[s * (SPT // GPW) + j], sidx_v)
            pltpu.async_copy(ones_v, obj_out.at[sidx_v], sem2).wait()

    for h in handles:
        h.wait()
    pltpu.sync_copy(rows_v, gath_out.at[wid])


@functools.lru_cache(maxsize=1)
def _get_sc_call():
    return pl.kernel(
        _sc_body,
        out_type=[
            jax.ShapeDtypeStruct((NW, CH, GPW), jnp.float32),
            jax.ShapeDtypeStruct((NPOS,), jnp.float32),
        ],
        mesh=plsc.VectorSubcoreMesh(core_axis_name="c", subcore_axis_name="s",
                                    num_cores=NC, num_subcores=NS),
        compiler_params=pltpu.CompilerParams(use_tc_tiling_on_sc=False),
        scratch_types=[
            pltpu.VMEM((CH, GPW), jnp.int32),
            pltpu.VMEM((CH, GPW), jnp.float32),
            pltpu.VMEM((GPW,), jnp.int32),
            pltpu.VMEM((GPW,), jnp.float32),
            pltpu.VMEM((ZPT,), jnp.float32),
            pltpu.SemaphoreType.DMA,
            pltpu.SemaphoreType.DMA,
        ],
    )


TI = 512
TJ = 1024
INV_NPOS = 1.0 / NPOS
INV_CLS = 1.0 / (N * NCLS)
INV_NN = 1.0 / (N * N)
ANG_S = 2.0 / 3.1416


def _atan(t):
    a = jnp.abs(t)
    inv = a > 1.0
    u = jnp.where(inv, 1.0 / a, a)
    u2 = u * u
    p = u * (0.9998660 + u2 * (-0.3302995 + u2 * (0.1801410
             + u2 * (-0.0851330 + u2 * 0.0208351))))
    r = jnp.where(inv, 1.5707964 - p, p)
    return jnp.where(t < 0.0, -r, r)


def _bce_sum(x, z):
    return jnp.sum(jnp.maximum(x, 0.0) - x * z
                   + jnp.log(1.0 + jnp.exp(-jnp.abs(x))))


def _prep(x1, y1, x2, y2):
    w = x2 - x1
    h = y2 - y1
    area = w * h
    cx = (x1 + x2) * 0.5
    cy = (y1 + y2) * 0.5
    ang = ANG_S * _atan(w / (h + EPS))
    return [x1, y1, x2, y2, w, h, area, cx, cy, ang]


NPAR = 10


def _fused_body(ox_ref, mz_ref, cx_ref, cz_ref, pr_ref, bt_ref, out_ref,
                *scr):
    pc = scr[:NPAR]
    pr_s = scr[NPAR:]
    k = pl.program_id(0)

    @pl.when(k == 0)
    def _():
        s_obj = _bce_sum(ox_ref[...], mz_ref[...])
        s_cls = _bce_sum(cx_ref[...], cz_ref[...])
        out_ref[...] = (s_obj * INV_NPOS + s_cls * INV_CLS
                        + 1.0).reshape(1, 1)
        cvals = _prep(pr_ref[:, 0:1], pr_ref[:, 1:2],
                      pr_ref[:, 2:3], pr_ref[:, 3:4])
        rvals = _prep(bt_ref[0:1, :], bt_ref[1:2, :],
                      bt_ref[2:3, :], bt_ref[3:4, :])
        for q in range(NPAR):
            pc[q][...] = cvals[q]
            pr_s[q][...] = rvals[q]

    @pl.when(k > 0)
    def _():
        kk = k - 1
        ib = kk // (N // TJ)
        jb = kk % (N // TJ)
        ci = ib * TI
        rj = jb * TJ
        (x1c, y1c, x2c, y2c, wc, hc, areac, cxc, cyc, angc) = [
            p[pl.ds(ci, TI), :] for p in pc]
        (x1r, y1r, x2r, y2r, wr, hr, arear, cxr, cyr, angr) = [
            p[:, pl.ds(rj, TJ)] for p in pr_s]

        ix1 = jnp.maximum(x1c, x1r)
        iy1 = jnp.maximum(y1c, y1r)
        ix2 = jnp.minimum(x2c, x2r)
        iy2 = jnp.minimum(y2c, y2r)
        iw = ix2 - ix1
        ih = iy2 - iy1
        inter = jnp.maximum(iw, 0.0) * jnp.maximum(ih, 0.0)
        union = (areac + arear) - inter
        iou = inter / (union + EPS)
        encw = (wc + wr) - iw
        ench = (hc + hr) - ih
        diag2 = encw * encw + (ench * ench + EPS)
        dx = cxc - cxr
        dy = cyc - cyr
        dist2 = dx * dx + dy * dy
        dv = angc - angr
        v = dv * dv
        denom = ((1.0 - iou) + v) + EPS
        ciou = (iou - dist2 / diag2) - (v / denom) * v
        out_ref[...] += (jnp.sum(ciou) * (-INV_NN)).reshape(1, 1)


def kernel(pred, box, cls, grid_x, grid_y, grid_anchor):
    f32 = jnp.float32
    flat = (grid_anchor.astype(jnp.int32) * (H * W)
            + grid_y.astype(jnp.int32) * W + grid_x.astype(jnp.int32))
    idx2 = flat.reshape(NW, GPW)
    widx3 = (flat[:, None] * CH
             + jnp.arange(CH, dtype=jnp.int32)[None, :]).reshape(NW, CH, GPW)
    pred1d = pred.reshape(NPOS * CH)
    ones_h = jnp.ones((GPW,), f32)
    zeros_h = jnp.zeros((ZPT,), f32)

    gath3, mask = _get_sc_call()(pred1d, idx2, widx3, ones_h, zeros_h)
    gath = gath3.reshape(N, CH)

    obj_x = pred[..., 4].reshape(NPOS // 128, 128)
    mask2 = mask.reshape(NPOS // 128, 128)
    boxt = jnp.zeros((8, N), f32).at[0:4, :].set(box.T)

    njt = N // TJ
    total = pl.pallas_call(
        _fused_body,
        grid=(1 + (N // TI) * njt,),
        in_specs=[
            pl.BlockSpec((NPOS // 128, 128), lambda k: (0, 0)),
            pl.BlockSpec((NPOS // 128, 128), lambda k: (0, 0)),
            pl.BlockSpec((N, NCLS), lambda k: (0, 0)),
            pl.BlockSpec((N, NCLS), lambda k: (0, 0)),
            pl.BlockSpec((N, 4), lambda k: (0, 0)),
            pl.BlockSpec((8, N), lambda k: (0, 0)),
        ],
        out_specs=pl.BlockSpec((1, 1), lambda k: (0, 0)),
        out_shape=jax.ShapeDtypeStruct((1, 1), f32),
        scratch_shapes=([pltpu.VMEM((N, 1), f32)] * 10
                        + [pltpu.VMEM((1, N), f32)] * 10),
    )(obj_x, mask2, gath[:, 5:CH], cls, gath[:, 0:4], boxt)
    return total[0, 0]


# speedup vs baseline: 1.3040x; 1.0102x over previous
import functools

import jax
import jax.numpy as jnp
from jax import lax
from jax.experimental import pallas as pl
from jax.experimental.pallas import tpu as pltpu
from jax.experimental.pallas import tpu_sc as plsc

A, H, W, CH = 3, 256, 256, 85
NPOS = A * H * W
N = 4096
NCLS = 80
NC, NS = 2, 16
NW = NC * NS
GPW = N // NW
SPT = N // NS
ZPT = NPOS // NS
EPS = 1e-07


def _sc_body(pred1d, idx2, widx3, ones_h, zeros_h, gath_out, obj_out,
             widx_v, rows_v, sidx_v, ones_v, zeros_v, sem, sem2):
    c = lax.axis_index("c")
    s = lax.axis_index("s")
    wid = s * NC + c
    pltpu.sync_copy(widx3.at[wid], widx_v)
    handles = [pltpu.async_copy(pred1d.at[widx_v.at[j]], rows_v.at[j], sem)
               for j in range(CH)]

    @pl.when(c == 0)
    def _():
        pltpu.sync_copy(zeros_h, zeros_v)
        pltpu.sync_copy(zeros_v, obj_out.at[pl.ds(s * ZPT, ZPT)])
        plsc.subcore_barrier()
        pltpu.sync_copy(ones_h, ones_v)
        for j in range(SPT // GPW):
            pltpu.sync_copy(idx2.at[s * (SPT // GPW) + j], sidx_v)
            pltpu.async_copy(ones_v, obj_out.at[sidx_v], sem2).wait()

    for h in handles:
        h.wait()
    pltpu.sync_copy(rows_v, gath_out.at[wid])


@functools.lru_cache(maxsize=1)
def _get_sc_call():
    return pl.kernel(
        _sc_body,
        out_type=[
            jax.ShapeDtypeStruct((NW, CH, GPW), jnp.float32),
            jax.ShapeDtypeStruct((NPOS,), jnp.float32),
        ],
        mesh=plsc.VectorSubcoreMesh(core_axis_name="c", subcore_axis_name="s",
                                    num_cores=NC, num_subcores=NS),
        compiler_params=pltpu.CompilerParams(use_tc_tiling_on_sc=False),
        scratch_types=[
            pltpu.VMEM((CH, GPW), jnp.int32),
            pltpu.VMEM((CH, GPW), jnp.float32),
            pltpu.VMEM((GPW,), jnp.int32),
            pltpu.VMEM((GPW,), jnp.float32),
            pltpu.VMEM((ZPT,), jnp.float32),
            pltpu.SemaphoreType.DMA,
            pltpu.SemaphoreType.DMA,
        ],
    )


TI = 512
TJ = 2048
INV_NPOS = 1.0 / NPOS
INV_CLS = 1.0 / (N * NCLS)
INV_NN = 1.0 / (N * N)
ANG_S = 2.0 / 3.1416


def _atan(t):
    a = jnp.abs(t)
    inv = a > 1.0
    u = jnp.where(inv, 1.0 / a, a)
    u2 = u * u
    p = u * (0.9998660 + u2 * (-0.3302995 + u2 * (0.1801410
             + u2 * (-0.0851330 + u2 * 0.0208351))))
    r = jnp.where(inv, 1.5707964 - p, p)
    return jnp.where(t < 0.0, -r, r)


def _bce_sum(x, z):
    return jnp.sum(jnp.maximum(x, 0.0) - x * z
                   + jnp.log(1.0 + jnp.exp(-jnp.abs(x))))


def _prep(x1, y1, x2, y2):
    w = x2 - x1
    h = y2 - y1
    area = w * h
    cx = (x1 + x2) * 0.5
    cy = (y1 + y2) * 0.5
    ang = ANG_S * _atan(w / (h + EPS))
    return [x1, y1, x2, y2, w, h, area, cx, cy, ang]


NPAR = 10


def _fused_body(ox_ref, mz_ref, cx_ref, cz_ref, pr_ref, bt_ref, out_ref,
                *scr):
    pc = scr[:NPAR]
    pr_s = scr[NPAR:]
    k = pl.program_id(0)

    @pl.when(k == 0)
    def _():
        s_obj = _bce_sum(ox_ref[...], mz_ref[...])
        s_cls = _bce_sum(cx_ref[...], cz_ref[...])
        out_ref[...] = (s_obj * INV_NPOS + s_cls * INV_CLS
                        + 1.0).reshape(1, 1)
        cvals = _prep(pr_ref[:, 0:1], pr_ref[:, 1:2],
                      pr_ref[:, 2:3], pr_ref[:, 3:4])
        rvals = _prep(bt_ref[0:1, :], bt_ref[1:2, :],
                      bt_ref[2:3, :], bt_ref[3:4, :])
        for q in range(NPAR):
            pc[q][...] = cvals[q]
            pr_s[q][...] = rvals[q]

    @pl.when(k > 0)
    def _():
        kk = k - 1
        ib = kk // (N // TJ)
        jb = kk % (N // TJ)
        ci = ib * TI
        rj = jb * TJ
        (x1c, y1c, x2c, y2c, wc, hc, areac, cxc, cyc, angc) = [
            p[pl.ds(ci, TI), :] for p in pc]
        (x1r, y1r, x2r, y2r, wr, hr, arear, cxr, cyr, angr) = [
            p[:, pl.ds(rj, TJ)] for p in pr_s]

        ix1 = jnp.maximum(x1c, x1r)
        iy1 = jnp.maximum(y1c, y1r)
        ix2 = jnp.minimum(x2c, x2r)
        iy2 = jnp.minimum(y2c, y2r)
        iw = ix2 - ix1
        ih = iy2 - iy1
        inter = jnp.maximum(iw, 0.0) * jnp.maximum(ih, 0.0)
        union = (areac + arear) - inter
        iou = inter / (union + EPS)
        encw = (wc + wr) - iw
        ench = (hc + hr) - ih
        diag2 = encw * encw + (ench * ench + EPS)
        dx = cxc - cxr
        dy = cyc - cyr
        dist2 = dx * dx + dy * dy
        dv = angc - angr
        v = dv * dv
        denom = ((1.0 - iou) + v) + EPS
        ciou = (iou - dist2 / diag2) - (v / denom) * v
        out_ref[...] += (jnp.sum(ciou) * (-INV_NN)).reshape(1, 1)


def kernel(pred, box, cls, grid_x, grid_y, grid_anchor):
    f32 = jnp.float32
    flat = (grid_anchor.astype(jnp.int32) * (H * W)
            + grid_y.astype(jnp.int32) * W + grid_x.astype(jnp.int32))
    idx2 = flat.reshape(NW, GPW)
    widx3 = (flat[:, None] * CH
             + jnp.arange(CH, dtype=jnp.int32)[None, :]).reshape(NW, CH, GPW)
    pred1d = pred.reshape(NPOS * CH)
    ones_h = jnp.ones((GPW,), f32)
    zeros_h = jnp.zeros((ZPT,), f32)

    gath3, mask = _get_sc_call()(pred1d, idx2, widx3, ones_h, zeros_h)
    gath = gath3.reshape(N, CH)

    obj_x = pred[..., 4].reshape(NPOS // 128, 128)
    mask2 = mask.reshape(NPOS // 128, 128)
    boxt = jnp.zeros((8, N), f32).at[0:4, :].set(box.T)

    njt = N // TJ
    total = pl.pallas_call(
        _fused_body,
        grid=(1 + (N // TI) * njt,),
        in_specs=[
            pl.BlockSpec((NPOS // 128, 128), lambda k: (0, 0)),
            pl.BlockSpec((NPOS // 128, 128), lambda k: (0, 0)),
            pl.BlockSpec((N, NCLS), lambda k: (0, 0)),
            pl.BlockSpec((N, NCLS), lambda k: (0, 0)),
            pl.BlockSpec((N, 4), lambda k: (0, 0)),
            pl.BlockSpec((8, N), lambda k: (0, 0)),
        ],
        out_specs=pl.BlockSpec((1, 1), lambda k: (0, 0)),
        out_shape=jax.ShapeDtypeStruct((1, 1), f32),
        scratch_shapes=([pltpu.VMEM((N, 1), f32)] * 10
                        + [pltpu.VMEM((1, N), f32)] * 10),
    )(obj_x, mask2, gath[:, 5:CH], cls, gath[:, 0:4], boxt)
    return total[0, 0]


# speedup vs baseline: 1.3056x; 1.0012x over previous
import functools

import jax
import jax.numpy as jnp
from jax import lax
from jax.experimental import pallas as pl
from jax.experimental.pallas import tpu as pltpu
from jax.experimental.pallas import tpu_sc as plsc

A, H, W, CH = 3, 256, 256, 85
NPOS = A * H * W
N = 4096
NCLS = 80
NC, NS = 2, 16
NW = NC * NS
GPW = N // NW
SPT = N // NS
ZPT = NPOS // NS
EPS = 1e-07


def _sc_body(pred1d, idx2, widx3, ones_h, zeros_h, gath_out, obj_out,
             widx_v, rows_v, sidx_v, ones_v, zeros_v, sem, sem2):
    c = lax.axis_index("c")
    s = lax.axis_index("s")
    wid = s * NC + c
    pltpu.sync_copy(widx3.at[wid], widx_v)
    handles = [pltpu.async_copy(pred1d.at[widx_v.at[j]], rows_v.at[j], sem)
               for j in range(CH)]

    @pl.when(c == 0)
    def _():
        pltpu.sync_copy(zeros_h, zeros_v)
        pltpu.sync_copy(zeros_v, obj_out.at[pl.ds(s * ZPT, ZPT)])
        plsc.subcore_barrier()
        pltpu.sync_copy(ones_h, ones_v)
        for j in range(SPT // GPW):
            pltpu.sync_copy(idx2.at[s * (SPT // GPW) + j], sidx_v)
            pltpu.async_copy(ones_v, obj_out.at[sidx_v], sem2).wait()

    for h in handles:
        h.wait()
    pltpu.sync_copy(rows_v, gath_out.at[wid])


@functools.lru_cache(maxsize=1)
def _get_sc_call():
    return pl.kernel(
        _sc_body,
        out_type=[
            jax.ShapeDtypeStruct((NW, CH, GPW), jnp.float32),
            jax.ShapeDtypeStruct((NPOS,), jnp.float32),
        ],
        mesh=plsc.VectorSubcoreMesh(core_axis_name="c", subcore_axis_name="s",
                                    num_cores=NC, num_subcores=NS),
        compiler_params=pltpu.CompilerParams(use_tc_tiling_on_sc=False),
        scratch_types=[
            pltpu.VMEM((CH, GPW), jnp.int32),
            pltpu.VMEM((CH, GPW), jnp.float32),
            pltpu.VMEM((GPW,), jnp.int32),
            pltpu.VMEM((GPW,), jnp.float32),
            pltpu.VMEM((ZPT,), jnp.float32),
            pltpu.SemaphoreType.DMA,
            pltpu.SemaphoreType.DMA,
        ],
    )


TI = 512
TJ = 2048
INV_NPOS = 1.0 / NPOS
INV_CLS = 1.0 / (N * NCLS)
INV_NN = 1.0 / (N * N)
ANG_S = 2.0 / 3.1416


def _atan(t):
    a = jnp.abs(t)
    inv = a > 1.0
    u = jnp.where(inv, 1.0 / a, a)
    u2 = u * u
    p = u * (0.9998660 + u2 * (-0.3302995 + u2 * (0.1801410
             + u2 * (-0.0851330 + u2 * 0.0208351))))
    r = jnp.where(inv, 1.5707964 - p, p)
    return jnp.where(t < 0.0, -r, r)


def _bce_sum(x, z):
    return jnp.sum(jnp.maximum(x, 0.0) - x * z
                   + jnp.log(1.0 + jnp.exp(-jnp.abs(x))))


def _prep(x1, y1, x2, y2):
    w = x2 - x1
    h = y2 - y1
    area = w * h
    cx = (x1 + x2) * 0.5
    cy = (y1 + y2) * 0.5
    ang = ANG_S * _atan(w / (h + EPS))
    return [x1, y1, x2, y2, w, h, area, cx, cy, ang]


NPAR = 10


def _objdense_body(x_ref, out_ref):
    x = x_ref[...]
    out_ref[...] = jnp.sum(jnp.maximum(x, 0.0)
                           + jnp.log(1.0 + jnp.exp(-jnp.abs(x)))
                           ).reshape(1, 1)


def _fused_body(od_ref, ox_ref, mz_ref, cx_ref, cz_ref, pr_ref, bt_ref,
                out_ref, *scr):
    pc = scr[:NPAR]
    pr_s = scr[NPAR:]
    k = pl.program_id(0)

    @pl.when(k == 0)
    def _():
        s_obj = od_ref[0, 0] - jnp.sum(ox_ref[...] * mz_ref[...])
        s_cls = _bce_sum(cx_ref[...], cz_ref[...])
        out_ref[...] = (s_obj * INV_NPOS + s_cls * INV_CLS
                        + 1.0).reshape(1, 1)
        cvals = _prep(pr_ref[:, 0:1], pr_ref[:, 1:2],
                      pr_ref[:, 2:3], pr_ref[:, 3:4])
        rvals = _prep(bt_ref[0:1, :], bt_ref[1:2, :],
                      bt_ref[2:3, :], bt_ref[3:4, :])
        for q in range(NPAR):
            pc[q][...] = cvals[q]
            pr_s[q][...] = rvals[q]

    @pl.when(k > 0)
    def _():
        kk = k - 1
        ib = kk // (N // TJ)
        jb = kk % (N // TJ)
        ci = ib * TI
        rj = jb * TJ
        (x1c, y1c, x2c, y2c, wc, hc, areac, cxc, cyc, angc) = [
            p[pl.ds(ci, TI), :] for p in pc]
        (x1r, y1r, x2r, y2r, wr, hr, arear, cxr, cyr, angr) = [
            p[:, pl.ds(rj, TJ)] for p in pr_s]

        ix1 = jnp.maximum(x1c, x1r)
        iy1 = jnp.maximum(y1c, y1r)
        ix2 = jnp.minimum(x2c, x2r)
        iy2 = jnp.minimum(y2c, y2r)
        iw = ix2 - ix1
        ih = iy2 - iy1
        inter = jnp.maximum(iw, 0.0) * jnp.maximum(ih, 0.0)
        union = (areac + arear) - inter
        iou = inter / (union + EPS)
        encw = (wc + wr) - iw
        ench = (hc + hr) - ih
        diag2 = encw * encw + (ench * ench + EPS)
        dx = cxc - cxr
        dy = cyc - cyr
        dist2 = dx * dx + dy * dy
        dv = angc - angr
        v = dv * dv
        denom = ((1.0 - iou) + v) + EPS
        ciou = (iou - dist2 / diag2) - (v / denom) * v
        out_ref[...] += (jnp.sum(ciou) * (-INV_NN)).reshape(1, 1)


def kernel(pred, box, cls, grid_x, grid_y, grid_anchor):
    f32 = jnp.float32
    flat = (grid_anchor.astype(jnp.int32) * (H * W)
            + grid_y.astype(jnp.int32) * W + grid_x.astype(jnp.int32))
    idx2 = flat.reshape(NW, GPW)
    widx3 = (flat[:, None] * CH
             + jnp.arange(CH, dtype=jnp.int32)[None, :]).reshape(NW, CH, GPW)
    pred1d = pred.reshape(NPOS * CH)
    ones_h = jnp.ones((GPW,), f32)
    zeros_h = jnp.zeros((ZPT,), f32)

    gath3, mask = _get_sc_call()(pred1d, idx2, widx3, ones_h, zeros_h)
    gath = gath3.reshape(N, CH)

    obj_x = pred[..., 4].reshape(NPOS // 128, 128)
    mask2 = mask.reshape(NPOS // 128, 128)
    boxt = jnp.zeros((8, N), f32).at[0:4, :].set(box.T)

    objd = pl.pallas_call(
        _objdense_body,
        out_shape=jax.ShapeDtypeStruct((1, 1), f32),
    )(obj_x)

    njt = N // TJ
    total = pl.pallas_call(
        _fused_body,
        grid=(1 + (N // TI) * njt,),
        in_specs=[
            pl.BlockSpec((1, 1), lambda k: (0, 0)),
            pl.BlockSpec((NPOS // 128, 128), lambda k: (0, 0)),
            pl.BlockSpec((NPOS // 128, 128), lambda k: (0, 0)),
            pl.BlockSpec((N, NCLS), lambda k: (0, 0)),
            pl.BlockSpec((N, NCLS), lambda k: (0, 0)),
            pl.BlockSpec((N, 4), lambda k: (0, 0)),
            pl.BlockSpec((8, N), lambda k: (0, 0)),
        ],
        out_specs=pl.BlockSpec((1, 1), lambda k: (0, 0)),
        out_shape=jax.ShapeDtypeStruct((1, 1), f32),
        scratch_shapes=([pltpu.VMEM((N, 1), f32)] * 10
                        + [pltpu.VMEM((1, N), f32)] * 10),
    )(objd, obj_x, mask2, gath[:, 5:CH], cls, gath[:, 0:4], boxt)
    return total[0, 0]
